# Initial kernel scaffold; baseline (speedup 1.0000x reference)
import jax
import jax.numpy as jnp
from jax.experimental import pallas as pl


def kernel(x, edge_index, batch, W1, b1, W2, b2, Wf, bf):
    raise NotImplementedError("write your pallas kernel here")


# speedup vs baseline: 9.0584x; 9.0584x over previous
import functools

import jax
import jax.numpy as jnp
from jax import lax
from jax.experimental import pallas as pl
from jax.experimental.pallas import tpu as pltpu
from jax.experimental.pallas import tpu_sc as plsc

N = 10000
E = 320000
DIN = 128
DH = 256
DHALF = 128
NG = 64
NSUB = 16
CHUNK = 80
DEGW = 16

def _deg_body(dst_hbm, out_hbm, didx, ones_v, deg_s, sem):
    c = lax.axis_index("c")
    s = lax.axis_index("s")
    one = jnp.ones((16,), jnp.float32)

    def fill(i, carry):
        ones_v[i, :] = one
        return carry

    lax.fori_loop(0, 104, fill, 0)

    nbase = s * 624
    for k in range(6):
        pltpu.sync_copy(ones_v, deg_s.at[pl.ds(nbase + k * 104, 104)])

    @pl.when(s == NSUB - 1)
    def _():
        pltpu.sync_copy(ones_v.at[pl.ds(0, 16)], deg_s.at[pl.ds(9984, 16)])

    plsc.subcore_barrier()

    wid = s * 2 + c
    e_per_w = E // (2 * NSUB)
    nch = e_per_w // CHUNK

    def body(i, carry):
        ebase = wid * e_per_w + i * CHUNK
        pltpu.sync_copy(dst_hbm.at[pl.ds(ebase, CHUNK)], didx)
        pltpu.sync_copy(ones_v.at[pl.ds(0, CHUNK)], deg_s.at[didx], add=True)
        return carry

    lax.fori_loop(0, nch, body, 0)
    plsc.subcore_barrier()
    pltpu.sync_copy(deg_s.at[pl.ds(nbase, 624)],
                    out_hbm.at[pl.ds(c * N + nbase, 624)])

    @pl.when(s == NSUB - 1)
    def _():
        pltpu.sync_copy(deg_s.at[pl.ds(9984, 16)],
                        out_hbm.at[pl.ds(c * N + 9984, 16)])


@functools.cache
def _get_deg():
    mesh = plsc.VectorSubcoreMesh(
        core_axis_name="c", subcore_axis_name="s",
        num_cores=2, num_subcores=NSUB)
    return pl.kernel(
        _deg_body,
        out_type=jax.ShapeDtypeStruct((2 * N, DEGW), jnp.float32),
        mesh=mesh,
        scratch_types=[
            pltpu.VMEM((CHUNK,), jnp.int32),
            pltpu.VMEM((104, DEGW), jnp.float32),
            pltpu.VMEM_SHARED((N, DEGW), jnp.float32),
            pltpu.SemaphoreType.DMA,
        ],
    )


def _mp_body(src_hbm, dst_hbm, y_hbm, out_hbm, sidx, didx, gidx, rows, acc,
             sem):
    c = lax.axis_index("c")
    s = lax.axis_index("s")
    tbl = c * N
    nbase = s * 624

    pltpu.sync_copy(y_hbm.at[pl.ds(tbl + nbase, 624)],
                    acc.at[pl.ds(nbase, 624)])

    @pl.when(s == NSUB - 1)
    def _():
        pltpu.sync_copy(y_hbm.at[pl.ds(tbl + 9984, 16)],
                        acc.at[pl.ds(9984, 16)])

    plsc.subcore_barrier()

    e_per_s = E // NSUB
    nch = e_per_s // CHUNK

    def body(i, carry):
        ebase = s * e_per_s + i * CHUNK
        pltpu.sync_copy(src_hbm.at[pl.ds(ebase, CHUNK)], sidx)
        pltpu.sync_copy(dst_hbm.at[pl.ds(ebase, CHUNK)], didx)
        for j in range(CHUNK // 16):
            sl = pl.ds(j * 16, 16)
            gidx[sl] = sidx[sl] + tbl
        pltpu.async_copy(y_hbm.at[gidx], rows, sem).wait()
        pltpu.sync_copy(rows, acc.at[didx], add=True)
        return carry

    lax.fori_loop(0, nch, body, 0)
    plsc.subcore_barrier()
    pltpu.sync_copy(acc.at[pl.ds(nbase, 624)],
                    out_hbm.at[pl.ds(tbl + nbase, 624)])

    @pl.when(s == NSUB - 1)
    def _():
        pltpu.sync_copy(acc.at[pl.ds(9984, 16)],
                        out_hbm.at[pl.ds(tbl + 9984, 16)])


@functools.cache
def _get_mp():
    mesh = plsc.VectorSubcoreMesh(
        core_axis_name="c", subcore_axis_name="s",
        num_cores=2, num_subcores=NSUB)
    return pl.kernel(
        _mp_body,
        out_type=jax.ShapeDtypeStruct((2 * N, DHALF), jnp.float32),
        mesh=mesh,
        scratch_types=[
            pltpu.VMEM((CHUNK,), jnp.int32),
            pltpu.VMEM((CHUNK,), jnp.int32),
            pltpu.VMEM((CHUNK,), jnp.int32),
            pltpu.VMEM((CHUNK, DHALF), jnp.float32),
            pltpu.VMEM_SHARED((N, DHALF), jnp.float32),
            pltpu.SemaphoreType.DMA,
        ],
    )


def _dinv(deg_ref):
    deg = (jnp.sum(deg_ref[:N, :], axis=1, keepdims=True)
           + jnp.sum(deg_ref[N:, :], axis=1, keepdims=True)) * (1.0 / DEGW)
    deg = deg - 1.0
    return lax.rsqrt(deg)


def _tc1_body(x_ref, w_ref, deg_ref, y_ref):
    dinv = _dinv(deg_ref)
    h = jnp.dot(x_ref[...], w_ref[...], preferred_element_type=jnp.float32)
    y = h * dinv
    y_ref[:N, :] = y[:, :DHALF]
    y_ref[N:, :] = y[:, DHALF:]


_tc1 = pl.pallas_call(
    _tc1_body,
    out_shape=jax.ShapeDtypeStruct((2 * N, DHALF), jnp.float32),
)


def _tc2_body(acc_ref, deg_ref, b_ref, w_ref, y_ref):
    dinv = _dinv(deg_ref)
    h = jnp.concatenate([acc_ref[:N, :], acc_ref[N:, :]], axis=1)
    o = jnp.maximum(h * dinv + b_ref[...], 0.0)
    h2 = jnp.dot(o, w_ref[...], preferred_element_type=jnp.float32)
    y = h2 * dinv
    y_ref[:N, :] = y[:, :DHALF]
    y_ref[N:, :] = y[:, DHALF:]


_tc2 = pl.pallas_call(
    _tc2_body,
    out_shape=jax.ShapeDtypeStruct((2 * N, DHALF), jnp.float32),
)


def _tc3_body(acc_ref, deg_ref, b_ref, batch_ref, wf_ref, bf_ref, out_ref):
    dinv = _dinv(deg_ref)
    h = jnp.concatenate([acc_ref[:N, :], acc_ref[N:, :]], axis=1)
    o2 = jnp.maximum(h * dinv + b_ref[...], 0.0)
    gids = lax.broadcasted_iota(jnp.int32, (NG, N), 0)
    oh = (gids == batch_ref[...]).astype(jnp.float32)
    cnt = jnp.sum(oh, axis=1, keepdims=True)
    ssum = jnp.dot(oh, o2, preferred_element_type=jnp.float32)
    smean = ssum / jnp.maximum(cnt, 1.0)
    pooled = jnp.concatenate([smean, ssum], axis=1)
    out_ref[...] = jnp.maximum(
        jnp.dot(pooled, wf_ref[...], preferred_element_type=jnp.float32)
        + bf_ref[...], 0.0)


_tc3 = pl.pallas_call(
    _tc3_body,
    out_shape=jax.ShapeDtypeStruct((NG, 1024), jnp.float32),
)


@jax.jit
def kernel(x, edge_index, batch, W1, b1, W2, b2, Wf, bf):
    src = edge_index[0]
    dst = edge_index[1]
    deg16 = _get_deg()(dst)
    y1 = _tc1(x, W1, deg16)
    acc1 = _get_mp()(src, dst, y1)
    y2 = _tc2(acc1, deg16, b1.reshape(1, -1), W2)
    acc2 = _get_mp()(src, dst, y2)
    return _tc3(acc2, deg16, b2.reshape(1, -1), batch.reshape(1, -1),
                Wf, bf.reshape(1, -1))


# speedup vs baseline: 15.1036x; 1.6674x over previous
import functools

import jax
import jax.numpy as jnp
from jax import lax
from jax.experimental import pallas as pl
from jax.experimental.pallas import tpu as pltpu
from jax.experimental.pallas import tpu_sc as plsc

N = 10000
E = 320000
DIN = 128
DH = 256
DHALF = 128
NG = 64
NSUB = 16
CHUNK = 80
DEGW = 16

def _deg_body(dst_hbm, out_hbm, didx, ones_v, deg_s, sem):
    c = lax.axis_index("c")
    s = lax.axis_index("s")
    one = jnp.ones((16,), jnp.float32)

    def fill(i, carry):
        ones_v[i, :] = one
        return carry

    lax.fori_loop(0, 104, fill, 0)

    nbase = s * 624
    for k in range(6):
        pltpu.sync_copy(ones_v, deg_s.at[pl.ds(nbase + k * 104, 104)])

    @pl.when(s == NSUB - 1)
    def _():
        pltpu.sync_copy(ones_v.at[pl.ds(0, 16)], deg_s.at[pl.ds(9984, 16)])

    plsc.subcore_barrier()

    wid = s * 2 + c
    e_per_w = E // (2 * NSUB)
    nch = e_per_w // CHUNK

    def body(i, carry):
        ebase = wid * e_per_w + i * CHUNK
        pltpu.sync_copy(dst_hbm.at[pl.ds(ebase, CHUNK)], didx)
        pltpu.sync_copy(ones_v.at[pl.ds(0, CHUNK)], deg_s.at[didx], add=True)
        return carry

    lax.fori_loop(0, nch, body, 0)
    plsc.subcore_barrier()
    pltpu.sync_copy(deg_s.at[pl.ds(nbase, 624)],
                    out_hbm.at[pl.ds(c * N + nbase, 624)])

    @pl.when(s == NSUB - 1)
    def _():
        pltpu.sync_copy(deg_s.at[pl.ds(9984, 16)],
                        out_hbm.at[pl.ds(c * N + 9984, 16)])


@functools.cache
def _get_deg():
    mesh = plsc.VectorSubcoreMesh(
        core_axis_name="c", subcore_axis_name="s",
        num_cores=2, num_subcores=NSUB)
    return pl.kernel(
        _deg_body,
        out_type=jax.ShapeDtypeStruct((2 * N, DEGW), jnp.float32),
        mesh=mesh,
        scratch_types=[
            pltpu.VMEM((CHUNK,), jnp.int32),
            pltpu.VMEM((104, DEGW), jnp.float32),
            pltpu.VMEM_SHARED((N, DEGW), jnp.float32),
            pltpu.SemaphoreType.DMA,
        ],
    )


_EPS = E // NSUB
_NCH = _EPS // CHUNK


_BLK_CH = 10
_NBLK = _NCH // _BLK_CH
_EBLK = _BLK_CH * CHUNK


def _mp_body(src_hbm, dst_hbm, y_hbm, out_hbm, gst, dstb, r0, r1, acc,
             semg0, semg1, sems0, sems1):
    c = lax.axis_index("c")
    s = lax.axis_index("s")
    tbl = c * N
    nbase = s * 624

    pltpu.sync_copy(y_hbm.at[pl.ds(tbl + nbase, 624)],
                    acc.at[pl.ds(nbase, 624)])

    @pl.when(s == NSUB - 1)
    def _():
        pltpu.sync_copy(y_hbm.at[pl.ds(tbl + 9984, 16)],
                        acc.at[pl.ds(9984, 16)])

    plsc.subcore_barrier()

    def block(b, carry):
        pltpu.sync_copy(src_hbm.at[pl.ds(s * _EPS + b * _EBLK, _EBLK)], gst)
        pltpu.sync_copy(dst_hbm.at[s, b], dstb)

        @pl.when(c == 1)
        def _():
            def adj(j, inner):
                sl = pl.ds(j * 16, 16)
                gst[sl] = gst[sl] + tbl
                return inner

            lax.fori_loop(0, _EBLK // 16, adj, 0)

        pltpu.async_copy(y_hbm.at[gst.at[pl.ds(0, CHUNK)]], r0, semg0).wait()

        def pair(p, inner):
            e = 2 * p
            cg1 = pltpu.async_copy(
                y_hbm.at[gst.at[pl.ds((e + 1) * CHUNK, CHUNK)]], r1, semg1)
            cs0 = pltpu.async_copy(r0, acc.at[dstb.at[e]], sems0, add=True)
            cg1.wait()
            cs1 = pltpu.async_copy(r1, acc.at[dstb.at[e + 1]], sems1,
                                   add=True)
            cs0.wait()

            @pl.when(e + 2 < _BLK_CH)
            def _():
                pltpu.async_copy(
                    y_hbm.at[gst.at[pl.ds((e + 2) * CHUNK, CHUNK)]],
                    r0, semg0)
                pltpu.make_async_copy(y_hbm.at[pl.ds(0, CHUNK)], r0,
                                      semg0).wait()

            cs1.wait()
            return inner

        lax.fori_loop(0, _BLK_CH // 2, pair, 0)
        return carry

    lax.fori_loop(0, _NBLK, block, 0)
    plsc.subcore_barrier()
    pltpu.sync_copy(acc.at[pl.ds(nbase, 624)],
                    out_hbm.at[pl.ds(tbl + nbase, 624)])

    @pl.when(s == NSUB - 1)
    def _():
        pltpu.sync_copy(acc.at[pl.ds(9984, 16)],
                        out_hbm.at[pl.ds(tbl + 9984, 16)])


@functools.cache
def _get_mp():
    mesh = plsc.VectorSubcoreMesh(
        core_axis_name="c", subcore_axis_name="s",
        num_cores=2, num_subcores=NSUB)
    return pl.kernel(
        _mp_body,
        out_type=jax.ShapeDtypeStruct((2 * N, DHALF), jnp.float32),
        mesh=mesh,
        scratch_types=[
            pltpu.VMEM((_EBLK,), jnp.int32),
            pltpu.VMEM((_BLK_CH, CHUNK), jnp.int32),
            pltpu.VMEM((CHUNK, DHALF), jnp.float32),
            pltpu.VMEM((CHUNK, DHALF), jnp.float32),
            pltpu.VMEM_SHARED((N, DHALF), jnp.float32),
            pltpu.SemaphoreType.DMA,
            pltpu.SemaphoreType.DMA,
            pltpu.SemaphoreType.DMA,
            pltpu.SemaphoreType.DMA,
        ],
    )


def _dinv(deg_ref):
    deg = (jnp.sum(deg_ref[:N, :], axis=1, keepdims=True)
           + jnp.sum(deg_ref[N:, :], axis=1, keepdims=True)) * (1.0 / DEGW)
    deg = deg - 1.0
    return lax.rsqrt(deg)


def _tc1_body(x_ref, w_ref, deg_ref, y_ref):
    dinv = _dinv(deg_ref)
    h = jnp.dot(x_ref[...], w_ref[...], preferred_element_type=jnp.float32)
    y = h * dinv
    y_ref[:N, :] = y[:, :DHALF]
    y_ref[N:, :] = y[:, DHALF:]


_tc1 = pl.pallas_call(
    _tc1_body,
    out_shape=jax.ShapeDtypeStruct((2 * N, DHALF), jnp.float32),
)


def _tc2_body(acc_ref, deg_ref, b_ref, w_ref, y_ref):
    dinv = _dinv(deg_ref)
    h = jnp.concatenate([acc_ref[:N, :], acc_ref[N:, :]], axis=1)
    o = jnp.maximum(h * dinv + b_ref[...], 0.0)
    h2 = jnp.dot(o, w_ref[...], preferred_element_type=jnp.float32)
    y = h2 * dinv
    y_ref[:N, :] = y[:, :DHALF]
    y_ref[N:, :] = y[:, DHALF:]


_tc2 = pl.pallas_call(
    _tc2_body,
    out_shape=jax.ShapeDtypeStruct((2 * N, DHALF), jnp.float32),
)


def _tc3_body(acc_ref, deg_ref, b_ref, batch_ref, wf_ref, bf_ref, out_ref):
    dinv = _dinv(deg_ref)
    h = jnp.concatenate([acc_ref[:N, :], acc_ref[N:, :]], axis=1)
    o2 = jnp.maximum(h * dinv + b_ref[...], 0.0)
    gids = lax.broadcasted_iota(jnp.int32, (NG, N), 0)
    oh = (gids == batch_ref[...]).astype(jnp.float32)
    cnt = jnp.sum(oh, axis=1, keepdims=True)
    ssum = jnp.dot(oh, o2, preferred_element_type=jnp.float32)
    smean = ssum / jnp.maximum(cnt, 1.0)
    pooled = jnp.concatenate([smean, ssum], axis=1)
    out_ref[...] = jnp.maximum(
        jnp.dot(pooled, wf_ref[...], preferred_element_type=jnp.float32)
        + bf_ref[...], 0.0)


_tc3 = pl.pallas_call(
    _tc3_body,
    out_shape=jax.ShapeDtypeStruct((NG, 1024), jnp.float32),
)


@jax.jit
def kernel(x, edge_index, batch, W1, b1, W2, b2, Wf, bf):
    src = edge_index[0]
    dst = edge_index[1]
    dst2 = dst.reshape(NSUB, _NBLK, _BLK_CH, CHUNK)
    deg16 = _get_deg()(dst)
    y1 = _tc1(x, W1, deg16)
    acc1 = _get_mp()(src, dst2, y1)
    y2 = _tc2(acc1, deg16, b1.reshape(1, -1), W2)
    acc2 = _get_mp()(src, dst2, y2)
    return _tc3(acc2, deg16, b2.reshape(1, -1), batch.reshape(1, -1),
                Wf, bf.reshape(1, -1))


# speedup vs baseline: 21.8382x; 1.4459x over previous
import functools

import jax
import jax.numpy as jnp
from jax import lax
from jax.experimental import pallas as pl
from jax.experimental.pallas import tpu as pltpu
from jax.experimental.pallas import tpu_sc as plsc

N = 10000
E = 320000
DIN = 128
DH = 256
DHALF = 128
NG = 64
NSUB = 16
CHUNK = 80
DEGW = 16

def _deg_body(dst_hbm, out_hbm, didx, ones_v, deg_s, sem):
    c = lax.axis_index("c")
    s = lax.axis_index("s")
    one = jnp.ones((16,), jnp.float32)

    def fill(i, carry):
        ones_v[i, :] = one
        return carry

    lax.fori_loop(0, 104, fill, 0)

    nbase = s * 624
    for k in range(6):
        pltpu.sync_copy(ones_v, deg_s.at[pl.ds(nbase + k * 104, 104)])

    @pl.when(s == NSUB - 1)
    def _():
        pltpu.sync_copy(ones_v.at[pl.ds(0, 16)], deg_s.at[pl.ds(9984, 16)])

    plsc.subcore_barrier()

    wid = s * 2 + c
    e_per_w = E // (2 * NSUB)
    nch = e_per_w // CHUNK

    def body(i, carry):
        ebase = wid * e_per_w + i * CHUNK
        pltpu.sync_copy(dst_hbm.at[pl.ds(ebase, CHUNK)], didx)
        pltpu.sync_copy(ones_v.at[pl.ds(0, CHUNK)], deg_s.at[didx], add=True)
        return carry

    lax.fori_loop(0, nch, body, 0)
    plsc.subcore_barrier()
    pltpu.sync_copy(deg_s.at[pl.ds(nbase, 624)],
                    out_hbm.at[pl.ds(c * N + nbase, 624)])

    @pl.when(s == NSUB - 1)
    def _():
        pltpu.sync_copy(deg_s.at[pl.ds(9984, 16)],
                        out_hbm.at[pl.ds(c * N + 9984, 16)])


@functools.cache
def _get_deg():
    mesh = plsc.VectorSubcoreMesh(
        core_axis_name="c", subcore_axis_name="s",
        num_cores=2, num_subcores=NSUB)
    return pl.kernel(
        _deg_body,
        out_type=jax.ShapeDtypeStruct((2 * N, DEGW), jnp.float32),
        mesh=mesh,
        scratch_types=[
            pltpu.VMEM((CHUNK,), jnp.int32),
            pltpu.VMEM((104, DEGW), jnp.float32),
            pltpu.VMEM_SHARED((N, DEGW), jnp.float32),
            pltpu.SemaphoreType.DMA,
        ],
    )


_EPS = E // NSUB
_NCH = _EPS // CHUNK


_BLK_CH = 25
_NBLK = _NCH // _BLK_CH
_EBLK = _BLK_CH * CHUNK
_NRING = 3


def _mp_body(src_hbm, dst_hbm, y_hbm, out_hbm, gst, dstb, r0, r1, r2, acc,
             semg0, semg1, semg2, sems0, sems1, sems2):
    c = lax.axis_index("c")
    s = lax.axis_index("s")
    tbl = c * N
    nbase = s * 624
    rbufs = (r0, r1, r2)
    semg = (semg0, semg1, semg2)
    sems = (sems0, sems1, sems2)

    pltpu.sync_copy(y_hbm.at[pl.ds(tbl + nbase, 624)],
                    acc.at[pl.ds(nbase, 624)])

    @pl.when(s == NSUB - 1)
    def _():
        pltpu.sync_copy(y_hbm.at[pl.ds(tbl + 9984, 16)],
                        acc.at[pl.ds(9984, 16)])

    plsc.subcore_barrier()

    def gather(e, ring):
        return pltpu.async_copy(
            y_hbm.at[gst.at[pl.ds(e * CHUNK, CHUNK)]], rbufs[ring],
            semg[ring])

    def scatter(e, ring):
        return pltpu.async_copy(rbufs[ring], acc.at[dstb.at[e]], sems[ring],
                                add=True)

    def drain_scatter(ring):
        pltpu.make_async_copy(y_hbm.at[pl.ds(0, CHUNK)], rbufs[ring],
                              sems[ring]).wait()

    def block(b, carry):
        pltpu.sync_copy(src_hbm.at[pl.ds(s * _EPS + b * _EBLK, _EBLK)], gst)
        pltpu.sync_copy(dst_hbm.at[s, b], dstb)

        @pl.when(c == 1)
        def _():
            def adj(j, inner):
                sl = pl.ds(j * 16, 16)
                gst[sl] = gst[sl] + tbl
                return inner

            lax.fori_loop(0, _EBLK // 16, adj, 0)

        gather(0, 0)
        gather(1, 1)
        for e in range(_BLK_CH):
            ring = e % _NRING
            pltpu.make_async_copy(y_hbm.at[pl.ds(0, CHUNK)], rbufs[ring],
                                  semg[ring]).wait()
            scatter(e, ring)
            if e + 2 < _BLK_CH:
                nring = (e + 2) % _NRING
                if e - 1 >= 0:
                    drain_scatter(nring)
                gather(e + 2, nring)
        drain_scatter((_BLK_CH - 3) % _NRING)
        drain_scatter((_BLK_CH - 2) % _NRING)
        drain_scatter((_BLK_CH - 1) % _NRING)
        return carry

    lax.fori_loop(0, _NBLK, block, 0)
    plsc.subcore_barrier()
    pltpu.sync_copy(acc.at[pl.ds(nbase, 624)],
                    out_hbm.at[pl.ds(tbl + nbase, 624)])

    @pl.when(s == NSUB - 1)
    def _():
        pltpu.sync_copy(acc.at[pl.ds(9984, 16)],
                        out_hbm.at[pl.ds(tbl + 9984, 16)])


@functools.cache
def _get_mp():
    mesh = plsc.VectorSubcoreMesh(
        core_axis_name="c", subcore_axis_name="s",
        num_cores=2, num_subcores=NSUB)
    return pl.kernel(
        _mp_body,
        out_type=jax.ShapeDtypeStruct((2 * N, DHALF), jnp.float32),
        mesh=mesh,
        scratch_types=[
            pltpu.VMEM((_EBLK,), jnp.int32),
            pltpu.VMEM((_BLK_CH, CHUNK), jnp.int32),
            pltpu.VMEM((CHUNK, DHALF), jnp.float32),
            pltpu.VMEM((CHUNK, DHALF), jnp.float32),
            pltpu.VMEM((CHUNK, DHALF), jnp.float32),
            pltpu.VMEM_SHARED((N, DHALF), jnp.float32),
            pltpu.SemaphoreType.DMA,
            pltpu.SemaphoreType.DMA,
            pltpu.SemaphoreType.DMA,
            pltpu.SemaphoreType.DMA,
            pltpu.SemaphoreType.DMA,
            pltpu.SemaphoreType.DMA,
        ],
    )


def _dinv(deg_ref):
    deg = (jnp.sum(deg_ref[:N, :], axis=1, keepdims=True)
           + jnp.sum(deg_ref[N:, :], axis=1, keepdims=True)) * (1.0 / DEGW)
    deg = deg - 1.0
    return lax.rsqrt(deg)


def _tc1_body(x_ref, w_ref, deg_ref, y_ref):
    dinv = _dinv(deg_ref)
    h = jnp.dot(x_ref[...], w_ref[...], preferred_element_type=jnp.float32)
    y = h * dinv
    y_ref[:N, :] = y[:, :DHALF]
    y_ref[N:, :] = y[:, DHALF:]


_tc1 = pl.pallas_call(
    _tc1_body,
    out_shape=jax.ShapeDtypeStruct((2 * N, DHALF), jnp.float32),
)


def _tc2_body(acc_ref, deg_ref, b_ref, w_ref, y_ref):
    dinv = _dinv(deg_ref)
    h = jnp.concatenate([acc_ref[:N, :], acc_ref[N:, :]], axis=1)
    o = jnp.maximum(h * dinv + b_ref[...], 0.0)
    h2 = jnp.dot(o, w_ref[...], preferred_element_type=jnp.float32)
    y = h2 * dinv
    y_ref[:N, :] = y[:, :DHALF]
    y_ref[N:, :] = y[:, DHALF:]


_tc2 = pl.pallas_call(
    _tc2_body,
    out_shape=jax.ShapeDtypeStruct((2 * N, DHALF), jnp.float32),
)


def _tc3_body(acc_ref, deg_ref, b_ref, batch_ref, wf_ref, bf_ref, out_ref):
    dinv = _dinv(deg_ref)
    h = jnp.concatenate([acc_ref[:N, :], acc_ref[N:, :]], axis=1)
    o2 = jnp.maximum(h * dinv + b_ref[...], 0.0)
    gids = lax.broadcasted_iota(jnp.int32, (NG, N), 0)
    oh = (gids == batch_ref[...]).astype(jnp.float32)
    cnt = jnp.sum(oh, axis=1, keepdims=True)
    ssum = jnp.dot(oh, o2, preferred_element_type=jnp.float32)
    smean = ssum / jnp.maximum(cnt, 1.0)
    pooled = jnp.concatenate([smean, ssum], axis=1)
    out_ref[...] = jnp.maximum(
        jnp.dot(pooled, wf_ref[...], preferred_element_type=jnp.float32)
        + bf_ref[...], 0.0)


_tc3 = pl.pallas_call(
    _tc3_body,
    out_shape=jax.ShapeDtypeStruct((NG, 1024), jnp.float32),
)


@jax.jit
def kernel(x, edge_index, batch, W1, b1, W2, b2, Wf, bf):
    src = edge_index[0]
    dst = edge_index[1]
    dst2 = dst.reshape(NSUB, _NBLK, _BLK_CH, CHUNK)
    deg16 = _get_deg()(dst)
    y1 = _tc1(x, W1, deg16)
    acc1 = _get_mp()(src, dst2, y1)
    y2 = _tc2(acc1, deg16, b1.reshape(1, -1), W2)
    acc2 = _get_mp()(src, dst2, y2)
    return _tc3(acc2, deg16, b2.reshape(1, -1), batch.reshape(1, -1),
                Wf, bf.reshape(1, -1))


# speedup vs baseline: 23.8409x; 1.0917x over previous
import functools

import jax
import jax.numpy as jnp
from jax import lax
from jax.experimental import pallas as pl
from jax.experimental.pallas import tpu as pltpu
from jax.experimental.pallas import tpu_sc as plsc

N = 10000
E = 320000
DIN = 128
DH = 256
DHALF = 128
NG = 64
NSUB = 16
CHUNK = 80
DEGW = 16

def _deg_body(dst_hbm, out_hbm, didx, ones_v, deg_s):
    c = lax.axis_index("c")
    s = lax.axis_index("s")
    one = jnp.ones((16,), jnp.float32)

    def fill(i, carry):
        ones_v[i, :] = one
        return carry

    lax.fori_loop(0, 104, fill, 0)

    nbase = s * 624
    for k in range(6):
        pltpu.sync_copy(ones_v, deg_s.at[pl.ds(nbase + k * 104, 104)])

    @pl.when(s == NSUB - 1)
    def _():
        pltpu.sync_copy(ones_v.at[pl.ds(0, 16)], deg_s.at[pl.ds(9984, 16)])

    plsc.subcore_barrier()

    wid = s * 2 + c
    nblk = 5

    def body(b, carry):
        pltpu.sync_copy(dst_hbm.at[wid, b], didx)
        for e in range(25):
            pltpu.sync_copy(ones_v.at[pl.ds(0, CHUNK)], deg_s.at[didx.at[e]],
                            add=True)
        return carry

    lax.fori_loop(0, nblk, body, 0)
    plsc.subcore_barrier()
    pltpu.sync_copy(deg_s.at[pl.ds(nbase, 624)],
                    out_hbm.at[pl.ds(c * N + nbase, 624)])

    @pl.when(s == NSUB - 1)
    def _():
        pltpu.sync_copy(deg_s.at[pl.ds(9984, 16)],
                        out_hbm.at[pl.ds(c * N + 9984, 16)])


@functools.cache
def _get_deg():
    mesh = plsc.VectorSubcoreMesh(
        core_axis_name="c", subcore_axis_name="s",
        num_cores=2, num_subcores=NSUB)
    return pl.kernel(
        _deg_body,
        out_type=jax.ShapeDtypeStruct((2 * N, DEGW), jnp.float32),
        mesh=mesh,
        scratch_types=[
            pltpu.VMEM((25, CHUNK), jnp.int32),
            pltpu.VMEM((104, DEGW), jnp.float32),
            pltpu.VMEM_SHARED((N, DEGW), jnp.float32),
        ],
    )


_EPS = E // NSUB
_NCH = _EPS // CHUNK


_BLK_CH = 25
_NBLK = _NCH // _BLK_CH
_EBLK = _BLK_CH * CHUNK
_NRING = 3


def _mp_body(src_hbm, dst_hbm, y_hbm, out_hbm, gst, dstb, r0, r1, r2, acc,
             semg0, semg1, semg2, sems0, sems1, sems2):
    c = lax.axis_index("c")
    s = lax.axis_index("s")
    tbl = c * N
    nbase = s * 624
    rbufs = (r0, r1, r2)
    semg = (semg0, semg1, semg2)
    sems = (sems0, sems1, sems2)

    pltpu.sync_copy(y_hbm.at[pl.ds(tbl + nbase, 624)],
                    acc.at[pl.ds(nbase, 624)])

    @pl.when(s == NSUB - 1)
    def _():
        pltpu.sync_copy(y_hbm.at[pl.ds(tbl + 9984, 16)],
                        acc.at[pl.ds(9984, 16)])

    plsc.subcore_barrier()

    def gather(e, ring):
        return pltpu.async_copy(
            y_hbm.at[gst.at[pl.ds(e * CHUNK, CHUNK)]], rbufs[ring],
            semg[ring])

    def scatter(e, ring):
        return pltpu.async_copy(rbufs[ring], acc.at[dstb.at[e]], sems[ring],
                                add=True)

    def drain_scatter(ring):
        pltpu.make_async_copy(y_hbm.at[pl.ds(0, CHUNK)], rbufs[ring],
                              sems[ring]).wait()

    def block(b, carry):
        pltpu.sync_copy(src_hbm.at[pl.ds(s * _EPS + b * _EBLK, _EBLK)], gst)
        pltpu.sync_copy(dst_hbm.at[s, b], dstb)

        @pl.when(c == 1)
        def _():
            def adj(j, inner):
                sl = pl.ds(j * 16, 16)
                gst[sl] = gst[sl] + tbl
                return inner

            lax.fori_loop(0, _EBLK // 16, adj, 0)

        gather(0, 0)
        gather(1, 1)
        for e in range(_BLK_CH):
            ring = e % _NRING
            pltpu.make_async_copy(y_hbm.at[pl.ds(0, CHUNK)], rbufs[ring],
                                  semg[ring]).wait()
            scatter(e, ring)
            if e + 2 < _BLK_CH:
                nring = (e + 2) % _NRING
                if e - 1 >= 0:
                    drain_scatter(nring)
                gather(e + 2, nring)
        drain_scatter((_BLK_CH - 3) % _NRING)
        drain_scatter((_BLK_CH - 2) % _NRING)
        drain_scatter((_BLK_CH - 1) % _NRING)
        return carry

    lax.fori_loop(0, _NBLK, block, 0)
    plsc.subcore_barrier()
    pltpu.sync_copy(acc.at[pl.ds(nbase, 624)],
                    out_hbm.at[pl.ds(tbl + nbase, 624)])

    @pl.when(s == NSUB - 1)
    def _():
        pltpu.sync_copy(acc.at[pl.ds(9984, 16)],
                        out_hbm.at[pl.ds(tbl + 9984, 16)])


@functools.cache
def _get_mp():
    mesh = plsc.VectorSubcoreMesh(
        core_axis_name="c", subcore_axis_name="s",
        num_cores=2, num_subcores=NSUB)
    return pl.kernel(
        _mp_body,
        out_type=jax.ShapeDtypeStruct((2 * N, DHALF), jnp.float32),
        mesh=mesh,
        scratch_types=[
            pltpu.VMEM((_EBLK,), jnp.int32),
            pltpu.VMEM((_BLK_CH, CHUNK), jnp.int32),
            pltpu.VMEM((CHUNK, DHALF), jnp.float32),
            pltpu.VMEM((CHUNK, DHALF), jnp.float32),
            pltpu.VMEM((CHUNK, DHALF), jnp.float32),
            pltpu.VMEM_SHARED((N, DHALF), jnp.float32),
            pltpu.SemaphoreType.DMA,
            pltpu.SemaphoreType.DMA,
            pltpu.SemaphoreType.DMA,
            pltpu.SemaphoreType.DMA,
            pltpu.SemaphoreType.DMA,
            pltpu.SemaphoreType.DMA,
        ],
    )


def _dinv(deg_ref):
    deg = (jnp.sum(deg_ref[:N, :], axis=1, keepdims=True)
           + jnp.sum(deg_ref[N:, :], axis=1, keepdims=True)) * (1.0 / DEGW)
    deg = deg - 1.0
    return lax.rsqrt(deg)


def _tc1_body(x_ref, w_ref, deg_ref, y_ref):
    dinv = _dinv(deg_ref)
    h = jnp.dot(x_ref[...], w_ref[...], preferred_element_type=jnp.float32)
    y = h * dinv
    y_ref[:N, :] = y[:, :DHALF]
    y_ref[N:, :] = y[:, DHALF:]


_tc1 = pl.pallas_call(
    _tc1_body,
    out_shape=jax.ShapeDtypeStruct((2 * N, DHALF), jnp.float32),
)


def _tc2_body(acc_ref, deg_ref, b_ref, w_ref, y_ref):
    dinv = _dinv(deg_ref)
    h = jnp.concatenate([acc_ref[:N, :], acc_ref[N:, :]], axis=1)
    o = jnp.maximum(h * dinv + b_ref[...], 0.0)
    h2 = jnp.dot(o, w_ref[...], preferred_element_type=jnp.float32)
    y = h2 * dinv
    y_ref[:N, :] = y[:, :DHALF]
    y_ref[N:, :] = y[:, DHALF:]


_tc2 = pl.pallas_call(
    _tc2_body,
    out_shape=jax.ShapeDtypeStruct((2 * N, DHALF), jnp.float32),
)


def _tc3_body(acc_ref, deg_ref, b_ref, batch_ref, wf_ref, bf_ref, out_ref):
    dinv = _dinv(deg_ref)
    h = jnp.concatenate([acc_ref[:N, :], acc_ref[N:, :]], axis=1)
    o2 = jnp.maximum(h * dinv + b_ref[...], 0.0)
    gids = lax.broadcasted_iota(jnp.int32, (NG, N), 0)
    oh = (gids == batch_ref[...]).astype(jnp.float32)
    cnt = jnp.sum(oh, axis=1, keepdims=True)
    ssum = jnp.dot(oh, o2, preferred_element_type=jnp.float32)
    smean = ssum / jnp.maximum(cnt, 1.0)
    pooled = jnp.concatenate([smean, ssum], axis=1)
    out_ref[...] = jnp.maximum(
        jnp.dot(pooled, wf_ref[...], preferred_element_type=jnp.float32)
        + bf_ref[...], 0.0)


_tc3 = pl.pallas_call(
    _tc3_body,
    out_shape=jax.ShapeDtypeStruct((NG, 1024), jnp.float32),
)


@jax.jit
def kernel(x, edge_index, batch, W1, b1, W2, b2, Wf, bf):
    src = edge_index[0]
    dst = edge_index[1]
    dst2 = dst.reshape(NSUB, _NBLK, _BLK_CH, CHUNK)
    deg16 = _get_deg()(dst.reshape(2 * NSUB, 5, 25, CHUNK))
    y1 = _tc1(x, W1, deg16)
    acc1 = _get_mp()(src, dst2, y1)
    y2 = _tc2(acc1, deg16, b1.reshape(1, -1), W2)
    acc2 = _get_mp()(src, dst2, y2)
    return _tc3(acc2, deg16, b2.reshape(1, -1), batch.reshape(1, -1),
                Wf, bf.reshape(1, -1))


# speedup vs baseline: 24.9099x; 1.0448x over previous
import functools

import jax
import jax.numpy as jnp
from jax import lax
from jax.experimental import pallas as pl
from jax.experimental.pallas import tpu as pltpu
from jax.experimental.pallas import tpu_sc as plsc

N = 10000
E = 320000
DIN = 128
DH = 256
DHALF = 128
NG = 64
NSUB = 16
CHUNK = 80
DEGW = 16

def _deg_body(dst_hbm, out_hbm, didx, ones_v, deg_s):
    c = lax.axis_index("c")
    s = lax.axis_index("s")
    one = jnp.ones((16,), jnp.float32)

    def fill(i, carry):
        ones_v[i, :] = one
        return carry

    lax.fori_loop(0, 104, fill, 0)

    nbase = s * 624
    for k in range(6):
        pltpu.sync_copy(ones_v, deg_s.at[pl.ds(nbase + k * 104, 104)])

    @pl.when(s == NSUB - 1)
    def _():
        pltpu.sync_copy(ones_v.at[pl.ds(0, 16)], deg_s.at[pl.ds(9984, 16)])

    plsc.subcore_barrier()

    wid = s * 2 + c
    nblk = 5

    def body(b, carry):
        pltpu.sync_copy(dst_hbm.at[wid, b], didx)
        for e in range(25):
            pltpu.sync_copy(ones_v.at[pl.ds(0, CHUNK)], deg_s.at[didx.at[e]],
                            add=True)
        return carry

    lax.fori_loop(0, nblk, body, 0)
    plsc.subcore_barrier()
    pltpu.sync_copy(deg_s.at[pl.ds(nbase, 624)],
                    out_hbm.at[pl.ds(c * N + nbase, 624)])

    @pl.when(s == NSUB - 1)
    def _():
        pltpu.sync_copy(deg_s.at[pl.ds(9984, 16)],
                        out_hbm.at[pl.ds(c * N + 9984, 16)])


@functools.cache
def _get_deg():
    mesh = plsc.VectorSubcoreMesh(
        core_axis_name="c", subcore_axis_name="s",
        num_cores=2, num_subcores=NSUB)
    return pl.kernel(
        _deg_body,
        out_type=jax.ShapeDtypeStruct((2 * N, DEGW), jnp.float32),
        mesh=mesh,
        scratch_types=[
            pltpu.VMEM((25, CHUNK), jnp.int32),
            pltpu.VMEM((104, DEGW), jnp.float32),
            pltpu.VMEM_SHARED((N, DEGW), jnp.float32),
        ],
    )


_EPS = E // NSUB
_NCH = _EPS // CHUNK


_BLK_CH = 50
_NBLK = _NCH // _BLK_CH
_EBLK = _BLK_CH * CHUNK
_NRING = 3


def _mp_body(src_hbm, dst_hbm, y_hbm, out_hbm, gst, dstb, r0, r1, r2, acc,
             semg0, semg1, semg2, sems0, sems1, sems2):
    c = lax.axis_index("c")
    s = lax.axis_index("s")
    tbl = c * N
    nbase = s * 624
    rbufs = (r0, r1, r2)
    semg = (semg0, semg1, semg2)
    sems = (sems0, sems1, sems2)

    pltpu.sync_copy(y_hbm.at[pl.ds(tbl + nbase, 624)],
                    acc.at[pl.ds(nbase, 624)])

    @pl.when(s == NSUB - 1)
    def _():
        pltpu.sync_copy(y_hbm.at[pl.ds(tbl + 9984, 16)],
                        acc.at[pl.ds(9984, 16)])

    plsc.subcore_barrier()

    def gather(e, ring):
        return pltpu.async_copy(
            y_hbm.at[gst.at[pl.ds(e * CHUNK, CHUNK)]], rbufs[ring],
            semg[ring])

    def scatter(e, ring):
        return pltpu.async_copy(rbufs[ring], acc.at[dstb.at[e]], sems[ring],
                                add=True)

    def drain_scatter(ring):
        pltpu.make_async_copy(y_hbm.at[pl.ds(0, CHUNK)], rbufs[ring],
                              sems[ring]).wait()

    def block(b, carry):
        pltpu.sync_copy(src_hbm.at[pl.ds(s * _EPS + b * _EBLK, _EBLK)], gst)
        pltpu.sync_copy(dst_hbm.at[s, b], dstb)

        @pl.when(c == 1)
        def _():
            def adj(j, inner):
                sl = pl.ds(j * 16, 16)
                gst[sl] = gst[sl] + tbl
                return inner

            lax.fori_loop(0, _EBLK // 16, adj, 0)

        gather(0, 0)
        gather(1, 1)
        for e in range(_BLK_CH):
            ring = e % _NRING
            pltpu.make_async_copy(y_hbm.at[pl.ds(0, CHUNK)], rbufs[ring],
                                  semg[ring]).wait()
            scatter(e, ring)
            if e + 2 < _BLK_CH:
                nring = (e + 2) % _NRING
                if e - 1 >= 0:
                    drain_scatter(nring)
                gather(e + 2, nring)
        drain_scatter((_BLK_CH - 3) % _NRING)
        drain_scatter((_BLK_CH - 2) % _NRING)
        drain_scatter((_BLK_CH - 1) % _NRING)
        return carry

    lax.fori_loop(0, _NBLK, block, 0)
    plsc.subcore_barrier()
    pltpu.sync_copy(acc.at[pl.ds(nbase, 624)],
                    out_hbm.at[pl.ds(tbl + nbase, 624)])

    @pl.when(s == NSUB - 1)
    def _():
        pltpu.sync_copy(acc.at[pl.ds(9984, 16)],
                        out_hbm.at[pl.ds(tbl + 9984, 16)])


@functools.cache
def _get_mp():
    mesh = plsc.VectorSubcoreMesh(
        core_axis_name="c", subcore_axis_name="s",
        num_cores=2, num_subcores=NSUB)
    return pl.kernel(
        _mp_body,
        out_type=jax.ShapeDtypeStruct((2 * N, DHALF), jnp.float32),
        mesh=mesh,
        scratch_types=[
            pltpu.VMEM((_EBLK,), jnp.int32),
            pltpu.VMEM((_BLK_CH, CHUNK), jnp.int32),
            pltpu.VMEM((CHUNK, DHALF), jnp.float32),
            pltpu.VMEM((CHUNK, DHALF), jnp.float32),
            pltpu.VMEM((CHUNK, DHALF), jnp.float32),
            pltpu.VMEM_SHARED((N, DHALF), jnp.float32),
            pltpu.SemaphoreType.DMA,
            pltpu.SemaphoreType.DMA,
            pltpu.SemaphoreType.DMA,
            pltpu.SemaphoreType.DMA,
            pltpu.SemaphoreType.DMA,
            pltpu.SemaphoreType.DMA,
        ],
    )


def _dinv(deg_ref):
    deg = (jnp.sum(deg_ref[:N, :], axis=1, keepdims=True)
           + jnp.sum(deg_ref[N:, :], axis=1, keepdims=True)) * (1.0 / DEGW)
    deg = deg - 1.0
    return lax.rsqrt(deg)


def _tc1_body(x_ref, w_ref, deg_ref, y_ref):
    dinv = _dinv(deg_ref)
    h = jnp.dot(x_ref[...], w_ref[...], preferred_element_type=jnp.float32)
    y = h * dinv
    y_ref[:N, :] = y[:, :DHALF]
    y_ref[N:, :] = y[:, DHALF:]


_tc1 = pl.pallas_call(
    _tc1_body,
    out_shape=jax.ShapeDtypeStruct((2 * N, DHALF), jnp.float32),
)


def _tc2_body(acc_ref, deg_ref, b_ref, w_ref, y_ref):
    dinv = _dinv(deg_ref)
    h = jnp.concatenate([acc_ref[:N, :], acc_ref[N:, :]], axis=1)
    o = jnp.maximum(h * dinv + b_ref[...], 0.0)
    h2 = jnp.dot(o, w_ref[...], preferred_element_type=jnp.float32)
    y = h2 * dinv
    y_ref[:N, :] = y[:, :DHALF]
    y_ref[N:, :] = y[:, DHALF:]


_tc2 = pl.pallas_call(
    _tc2_body,
    out_shape=jax.ShapeDtypeStruct((2 * N, DHALF), jnp.float32),
)


def _tc3_body(acc_ref, deg_ref, b_ref, batch_ref, wf_ref, bf_ref, out_ref):
    dinv = _dinv(deg_ref)
    h = jnp.concatenate([acc_ref[:N, :], acc_ref[N:, :]], axis=1)
    o2 = jnp.maximum(h * dinv + b_ref[...], 0.0)
    gids = lax.broadcasted_iota(jnp.int32, (NG, N), 0)
    oh = (gids == batch_ref[...]).astype(jnp.float32)
    cnt = jnp.sum(oh, axis=1, keepdims=True)
    ssum = jnp.dot(oh, o2, preferred_element_type=jnp.float32)
    smean = ssum / jnp.maximum(cnt, 1.0)
    pooled = jnp.concatenate([smean, ssum], axis=1)
    out_ref[...] = jnp.maximum(
        jnp.dot(pooled, wf_ref[...], preferred_element_type=jnp.float32)
        + bf_ref[...], 0.0)


_tc3 = pl.pallas_call(
    _tc3_body,
    out_shape=jax.ShapeDtypeStruct((NG, 1024), jnp.float32),
)


@jax.jit
def kernel(x, edge_index, batch, W1, b1, W2, b2, Wf, bf):
    src = edge_index[0]
    dst = edge_index[1]
    dst2 = dst.reshape(NSUB, _NBLK, _BLK_CH, CHUNK)
    deg16 = _get_deg()(dst.reshape(2 * NSUB, 5, 25, CHUNK))
    y1 = _tc1(x, W1, deg16)
    acc1 = _get_mp()(src, dst2, y1)
    y2 = _tc2(acc1, deg16, b1.reshape(1, -1), W2)
    acc2 = _get_mp()(src, dst2, y2)
    return _tc3(acc2, deg16, b2.reshape(1, -1), batch.reshape(1, -1),
                Wf, bf.reshape(1, -1))


# speedup vs baseline: 25.6081x; 1.0280x over previous
import functools

import jax
import jax.numpy as jnp
from jax import lax
from jax.experimental import pallas as pl
from jax.experimental.pallas import tpu as pltpu
from jax.experimental.pallas import tpu_sc as plsc

N = 10000
E = 320000
DIN = 128
DH = 256
DHALF = 128
NG = 64
NSUB = 16
CHUNK = 80
DEGW = 16

def _deg_body(dst_hbm, out_hbm, didx, ones_v, deg_s):
    c = lax.axis_index("c")
    s = lax.axis_index("s")
    one = jnp.ones((16,), jnp.float32)

    def fill(i, carry):
        ones_v[i, :] = one
        return carry

    lax.fori_loop(0, 104, fill, 0)

    nbase = s * 624
    for k in range(6):
        pltpu.sync_copy(ones_v, deg_s.at[pl.ds(nbase + k * 104, 104)])

    @pl.when(s == NSUB - 1)
    def _():
        pltpu.sync_copy(ones_v.at[pl.ds(0, 16)], deg_s.at[pl.ds(9984, 16)])

    plsc.subcore_barrier()

    wid = s * 2 + c
    nblk = 5

    def body(b, carry):
        pltpu.sync_copy(dst_hbm.at[wid, b], didx)
        for e in range(25):
            pltpu.sync_copy(ones_v.at[pl.ds(0, CHUNK)], deg_s.at[didx.at[e]],
                            add=True)
        return carry

    lax.fori_loop(0, nblk, body, 0)
    plsc.subcore_barrier()
    pltpu.sync_copy(deg_s.at[pl.ds(nbase, 624)],
                    out_hbm.at[pl.ds(c * N + nbase, 624)])

    @pl.when(s == NSUB - 1)
    def _():
        pltpu.sync_copy(deg_s.at[pl.ds(9984, 16)],
                        out_hbm.at[pl.ds(c * N + 9984, 16)])


@functools.cache
def _get_deg():
    mesh = plsc.VectorSubcoreMesh(
        core_axis_name="c", subcore_axis_name="s",
        num_cores=2, num_subcores=NSUB)
    return pl.kernel(
        _deg_body,
        out_type=jax.ShapeDtypeStruct((2 * N, DEGW), jnp.float32),
        mesh=mesh,
        scratch_types=[
            pltpu.VMEM((25, CHUNK), jnp.int32),
            pltpu.VMEM((104, DEGW), jnp.float32),
            pltpu.VMEM_SHARED((N, DEGW), jnp.float32),
        ],
    )


_EPS = E // NSUB
_NCH = _EPS // CHUNK


_BLK_CH = 50
_NBLK = _NCH // _BLK_CH
_EBLK = _BLK_CH * CHUNK
_NRING = 3


def _mp_body(src_hbm, dst_hbm, y_hbm, out_hbm, gst, dstb, r0, r1, r2, acc,
             semg0, semg1, semg2, sems0, sems1, sems2):
    c = lax.axis_index("c")
    s = lax.axis_index("s")
    tbl = c * N
    nbase = s * 624
    rbufs = (r0, r1, r2)
    semg = (semg0, semg1, semg2)
    sems = (sems0, sems1, sems2)

    ci = pltpu.async_copy(y_hbm.at[pl.ds(tbl + nbase, 624)],
                          acc.at[pl.ds(nbase, 624)], semg2)

    @pl.when(s == NSUB - 1)
    def _():
        pltpu.async_copy(y_hbm.at[pl.ds(tbl + 9984, 16)],
                         acc.at[pl.ds(9984, 16)], semg2)

    cs = pltpu.async_copy(src_hbm.at[c * NSUB + s, 0], gst, semg0)
    cd = pltpu.async_copy(dst_hbm.at[s, 0], dstb, semg1)
    ci.wait()

    @pl.when(s == NSUB - 1)
    def _():
        pltpu.make_async_copy(y_hbm.at[pl.ds(9984, 16)],
                              acc.at[pl.ds(9984, 16)], semg2).wait()

    cs.wait()
    cd.wait()
    plsc.subcore_barrier()

    def gather(e, ring):
        return pltpu.async_copy(
            y_hbm.at[gst.at[e]], rbufs[ring],
            semg[ring])

    def scatter(e, ring):
        return pltpu.async_copy(rbufs[ring], acc.at[dstb.at[e]], sems[ring],
                                add=True)

    def drain_scatter(ring):
        pltpu.make_async_copy(y_hbm.at[pl.ds(0, CHUNK)], rbufs[ring],
                              sems[ring]).wait()

    def block(b, carry):
        @pl.when(b > 0)
        def _():
            ca = pltpu.async_copy(
                src_hbm.at[c * NSUB + s, b], gst, semg0)
            cb = pltpu.async_copy(dst_hbm.at[s, b], dstb, semg1)
            ca.wait()
            cb.wait()

        gather(0, 0)
        gather(1, 1)
        for e in range(_BLK_CH):
            ring = e % _NRING
            pltpu.make_async_copy(y_hbm.at[pl.ds(0, CHUNK)], rbufs[ring],
                                  semg[ring]).wait()
            scatter(e, ring)
            if e + 2 < _BLK_CH:
                nring = (e + 2) % _NRING
                if e - 1 >= 0:
                    drain_scatter(nring)
                gather(e + 2, nring)
        drain_scatter((_BLK_CH - 3) % _NRING)
        drain_scatter((_BLK_CH - 2) % _NRING)
        drain_scatter((_BLK_CH - 1) % _NRING)
        return carry

    lax.fori_loop(0, _NBLK, block, 0)
    plsc.subcore_barrier()
    pltpu.sync_copy(acc.at[pl.ds(nbase, 624)],
                    out_hbm.at[pl.ds(tbl + nbase, 624)])

    @pl.when(s == NSUB - 1)
    def _():
        pltpu.sync_copy(acc.at[pl.ds(9984, 16)],
                        out_hbm.at[pl.ds(tbl + 9984, 16)])


@functools.cache
def _get_mp():
    mesh = plsc.VectorSubcoreMesh(
        core_axis_name="c", subcore_axis_name="s",
        num_cores=2, num_subcores=NSUB)
    return pl.kernel(
        _mp_body,
        out_type=jax.ShapeDtypeStruct((2 * N, DHALF), jnp.float32),
        mesh=mesh,
        scratch_types=[
            pltpu.VMEM((_BLK_CH, CHUNK), jnp.int32),
            pltpu.VMEM((_BLK_CH, CHUNK), jnp.int32),
            pltpu.VMEM((CHUNK, DHALF), jnp.float32),
            pltpu.VMEM((CHUNK, DHALF), jnp.float32),
            pltpu.VMEM((CHUNK, DHALF), jnp.float32),
            pltpu.VMEM_SHARED((N, DHALF), jnp.float32),
            pltpu.SemaphoreType.DMA,
            pltpu.SemaphoreType.DMA,
            pltpu.SemaphoreType.DMA,
            pltpu.SemaphoreType.DMA,
            pltpu.SemaphoreType.DMA,
            pltpu.SemaphoreType.DMA,
        ],
    )


def _dinv(deg_ref):
    deg = (jnp.sum(deg_ref[:N, :], axis=1, keepdims=True)
           + jnp.sum(deg_ref[N:, :], axis=1, keepdims=True)) * (1.0 / DEGW)
    deg = deg - 1.0
    return lax.rsqrt(deg)


def _tc1_body(x_ref, w_ref, deg_ref, y_ref):
    dinv = _dinv(deg_ref)
    h = jnp.dot(x_ref[...], w_ref[...], preferred_element_type=jnp.float32)
    y = h * dinv
    y_ref[:N, :] = y[:, :DHALF]
    y_ref[N:, :] = y[:, DHALF:]


_tc1 = pl.pallas_call(
    _tc1_body,
    out_shape=jax.ShapeDtypeStruct((2 * N, DHALF), jnp.float32),
)


def _tc2_body(acc_ref, deg_ref, b_ref, w_ref, y_ref):
    dinv = _dinv(deg_ref)
    h = jnp.concatenate([acc_ref[:N, :], acc_ref[N:, :]], axis=1)
    o = jnp.maximum(h * dinv + b_ref[...], 0.0)
    h2 = jnp.dot(o, w_ref[...], preferred_element_type=jnp.float32)
    y = h2 * dinv
    y_ref[:N, :] = y[:, :DHALF]
    y_ref[N:, :] = y[:, DHALF:]


_tc2 = pl.pallas_call(
    _tc2_body,
    out_shape=jax.ShapeDtypeStruct((2 * N, DHALF), jnp.float32),
)


def _tc3_body(acc_ref, deg_ref, b_ref, batch_ref, wf_ref, bf_ref, out_ref):
    dinv = _dinv(deg_ref)
    h = jnp.concatenate([acc_ref[:N, :], acc_ref[N:, :]], axis=1)
    o2 = jnp.maximum(h * dinv + b_ref[...], 0.0)
    gids = lax.broadcasted_iota(jnp.int32, (NG, N), 0)
    oh = (gids == batch_ref[...]).astype(jnp.float32)
    cnt = jnp.sum(oh, axis=1, keepdims=True)
    ssum = jnp.dot(oh, o2, preferred_element_type=jnp.float32)
    smean = ssum / jnp.maximum(cnt, 1.0)
    pooled = jnp.concatenate([smean, ssum], axis=1)
    out_ref[...] = jnp.maximum(
        jnp.dot(pooled, wf_ref[...], preferred_element_type=jnp.float32)
        + bf_ref[...], 0.0)


_tc3 = pl.pallas_call(
    _tc3_body,
    out_shape=jax.ShapeDtypeStruct((NG, 1024), jnp.float32),
)


@jax.jit
def kernel(x, edge_index, batch, W1, b1, W2, b2, Wf, bf):
    src = edge_index[0]
    dst = edge_index[1]
    dst2 = dst.reshape(NSUB, _NBLK, _BLK_CH, CHUNK)
    src2 = jnp.concatenate([src, src + N]).reshape(
        2 * NSUB, _NBLK, _BLK_CH, CHUNK)
    deg16 = _get_deg()(dst.reshape(2 * NSUB, 5, 25, CHUNK))
    y1 = _tc1(x, W1, deg16)
    acc1 = _get_mp()(src2, dst2, y1)
    y2 = _tc2(acc1, deg16, b1.reshape(1, -1), W2)
    acc2 = _get_mp()(src2, dst2, y2)
    return _tc3(acc2, deg16, b2.reshape(1, -1), batch.reshape(1, -1),
                Wf, bf.reshape(1, -1))
